# baseline (speedup 1.0000x reference)
import functools

import jax
import jax.numpy as jnp
from jax import lax
from jax.experimental import pallas as pl
from jax.experimental.pallas import tpu as pltpu
from jax.experimental.pallas import tpu_sc as plsc

G = 8
K_PER_G = 8
G_ACTIVE = 2
K_PER_G_ACTIVE = 2
NE = G + K_PER_G

_TILE_M = 2048
_TILE_P = 2048

_DIMS = (((1,), (1,)), ((), ()))


def _mm_body(x1_ref, x2_ref, w1a_ref, w1b_ref, w2a_ref, w2b_ref, o_ref):
    sg = (lax.dot_general(x1_ref[...], w1a_ref[...], _DIMS,
                          preferred_element_type=jnp.float32)
          + lax.dot_general(x2_ref[...], w1b_ref[...], _DIMS,
                            preferred_element_type=jnp.float32))
    se = (lax.dot_general(x1_ref[...], w2a_ref[...], _DIMS,
                          preferred_element_type=jnp.float32)
          + lax.dot_general(x2_ref[...], w2b_ref[...], _DIMS,
                            preferred_element_type=jnp.float32))
    o_ref[:G, :] = sg.T
    o_ref[G:, :] = se.T


def _scores_tc(x, w1, w2, chunk, n_chunks):
    m, k = x.shape
    mc = m // n_chunks
    kh = k // 2
    grid = (mc // _TILE_M,)
    off = chunk * (mc // _TILE_M)
    return pl.pallas_call(
        _mm_body,
        grid=grid,
        in_specs=[
            pl.BlockSpec((_TILE_M, kh), lambda i: (i + off, 0)),
            pl.BlockSpec((_TILE_M, kh), lambda i: (i + off, 1)),
            pl.BlockSpec((G, kh), lambda i: (0, 0)),
            pl.BlockSpec((G, kh), lambda i: (0, 1)),
            pl.BlockSpec((K_PER_G, kh), lambda i: (0, 0)),
            pl.BlockSpec((K_PER_G, kh), lambda i: (0, 1)),
        ],
        out_specs=pl.BlockSpec((NE, _TILE_M), lambda i: (0, i)),
        out_shape=jax.ShapeDtypeStruct((NE, mc), jnp.float32),
    )(x, x, w1, w1, w2, w2)


def _top2_of_8(cols):
    best = cols[0]
    bidx = jnp.zeros((16,), jnp.int32)
    sec = jnp.full((16,), -jnp.inf, jnp.float32)
    sidx = jnp.zeros((16,), jnp.int32)
    for e in range(1, 8):
        ge = cols[e]
        ev = jnp.full((16,), e, jnp.int32)
        gt_b = ge > best
        gt_s = ge > sec
        sec = jnp.where(gt_b, best, jnp.where(gt_s, ge, sec))
        sidx = jnp.where(gt_b, bidx, jnp.where(gt_s, ev, sidx))
        best = jnp.where(gt_b, ge, best)
        bidx = jnp.where(gt_b, ev, bidx)
    return best, bidx, sec, sidx


def _make_router_sc(n_tokens):
    info = plsc.get_sparse_core_info()
    nc, ns = info.num_cores, info.num_subcores
    nw = nc * ns
    tpw = n_tokens // nw
    nblk = tpw // 16
    mesh = plsc.VectorSubcoreMesh(core_axis_name="c", subcore_axis_name="s")

    @functools.partial(
        pl.kernel,
        mesh=mesh,
        out_type=[
            jax.ShapeDtypeStruct((4, n_tokens), jnp.int32),
            jax.ShapeDtypeStruct((4, n_tokens), jnp.float32),
        ],
        scratch_types=[
            pltpu.VMEM((NE, tpw), jnp.float32),
            pltpu.VMEM((4, tpw), jnp.int32),
            pltpu.VMEM((4, tpw), jnp.float32),
        ],
        compiler_params=pltpu.CompilerParams(needs_layout_passes=False),
    )
    def router(scores_hbm, idx_hbm, probs_hbm, scores_v, idx_v, probs_v):
        wid = lax.axis_index("s") * nc + lax.axis_index("c")
        base = wid * tpw
        pltpu.sync_copy(scores_hbm.at[:, pl.ds(base, tpw)], scores_v)

        def step(t, carry):
            sl = pl.ds(t * 16, 16)
            cols = [scores_v[e, sl] for e in range(NE)]
            gs1, gi1, gs2, gi2 = _top2_of_8(cols[:G])
            es1, ei1, es2, ei2 = _top2_of_8(cols[G:])
            idxs = [gi1 * K_PER_G + ei1, gi1 * K_PER_G + ei2,
                    gi2 * K_PER_G + ei1, gi2 * K_PER_G + ei2]
            cs = [gs1 + es1, gs1 + es2, gs2 + es1, gs2 + es2]
            mx = jnp.maximum(jnp.maximum(cs[0], cs[1]),
                             jnp.maximum(cs[2], cs[3]))
            es = [jnp.exp(c - mx) for c in cs]
            tot = (es[0] + es[1]) + (es[2] + es[3])
            for j in range(4):
                idx_v[j, sl] = idxs[j]
                probs_v[j, sl] = es[j] / tot
            return carry

        lax.fori_loop(0, nblk, step, 0)
        pltpu.sync_copy(idx_v, idx_hbm.at[:, pl.ds(base, tpw)])
        pltpu.sync_copy(probs_v, probs_hbm.at[:, pl.ds(base, tpw)])

    return router


def _pack_body(i_ref, p_ref, oi_ref, op_ref):
    oi_ref[...] = i_ref[...].T
    op_ref[...] = p_ref[...].T


def _pack_tc(idx_t, probs_t):
    n = idx_t.shape[1]
    grid = (n // _TILE_P,)
    return pl.pallas_call(
        _pack_body,
        grid=grid,
        in_specs=[
            pl.BlockSpec((4, _TILE_P), lambda i: (0, i)),
            pl.BlockSpec((4, _TILE_P), lambda i: (0, i)),
        ],
        out_specs=[
            pl.BlockSpec((_TILE_P, 4), lambda i: (i, 0)),
            pl.BlockSpec((_TILE_P, 4), lambda i: (i, 0)),
        ],
        out_shape=[
            jax.ShapeDtypeStruct((n, 4), jnp.int32),
            jax.ShapeDtypeStruct((n, 4), jnp.float32),
        ],
    )(idx_t, probs_t)


_N_CHUNKS = 2


def kernel(x, W1, W2):
    n_tokens = x.shape[0]
    router = _make_router_sc(n_tokens // _N_CHUNKS)
    idx_parts, probs_parts = [], []
    for c in range(_N_CHUNKS):
        scores_t = _scores_tc(x, W1, W2, c, _N_CHUNKS)
        idx_t, probs_t = router(scores_t)
        idx_parts.append(idx_t.T)
        probs_parts.append(probs_t.T)
    return (jnp.concatenate(idx_parts, axis=0),
            jnp.concatenate(probs_parts, axis=0))


# speedup vs baseline: 1.1507x; 1.1507x over previous
import functools

import jax
import jax.numpy as jnp
from jax import lax
from jax.experimental import pallas as pl
from jax.experimental.pallas import tpu as pltpu
from jax.experimental.pallas import tpu_sc as plsc

G = 8
K_PER_G = 8
G_ACTIVE = 2
K_PER_G_ACTIVE = 2
NE = G + K_PER_G

_TILE_M = 2048
_TILE_P = 2048

_DIMS = (((1,), (1,)), ((), ()))


def _mm_body(x1_ref, x2_ref, w1a_ref, w1b_ref, w2a_ref, w2b_ref, o_ref):
    sg = (lax.dot_general(w1a_ref[...], x1_ref[...], _DIMS,
                          preferred_element_type=jnp.float32)
          + lax.dot_general(w1b_ref[...], x2_ref[...], _DIMS,
                            preferred_element_type=jnp.float32))
    se = (lax.dot_general(w2a_ref[...], x1_ref[...], _DIMS,
                          preferred_element_type=jnp.float32)
          + lax.dot_general(w2b_ref[...], x2_ref[...], _DIMS,
                            preferred_element_type=jnp.float32))
    o_ref[:G, :] = sg
    o_ref[G:, :] = se


def _scores_tc(x, w1, w2, chunk, n_chunks):
    m, k = x.shape
    mc = m // n_chunks
    kh = k // 2
    grid = (mc // _TILE_M,)
    off = chunk * (mc // _TILE_M)
    return pl.pallas_call(
        _mm_body,
        grid=grid,
        in_specs=[
            pl.BlockSpec((_TILE_M, kh), lambda i: (i + off, 0)),
            pl.BlockSpec((_TILE_M, kh), lambda i: (i + off, 1)),
            pl.BlockSpec((G, kh), lambda i: (0, 0)),
            pl.BlockSpec((G, kh), lambda i: (0, 1)),
            pl.BlockSpec((K_PER_G, kh), lambda i: (0, 0)),
            pl.BlockSpec((K_PER_G, kh), lambda i: (0, 1)),
        ],
        out_specs=pl.BlockSpec((NE, _TILE_M), lambda i: (0, i)),
        out_shape=jax.ShapeDtypeStruct((NE, mc), jnp.float32),
    )(x, x, w1, w1, w2, w2)


def _top2_of_8(cols):
    best = cols[0]
    bidx = jnp.zeros((16,), jnp.int32)
    sec = jnp.full((16,), -jnp.inf, jnp.float32)
    sidx = jnp.zeros((16,), jnp.int32)
    for e in range(1, 8):
        ge = cols[e]
        ev = jnp.full((16,), e, jnp.int32)
        gt_b = ge > best
        gt_s = ge > sec
        sec = jnp.where(gt_b, best, jnp.where(gt_s, ge, sec))
        sidx = jnp.where(gt_b, bidx, jnp.where(gt_s, ev, sidx))
        best = jnp.where(gt_b, ge, best)
        bidx = jnp.where(gt_b, ev, bidx)
    return best, bidx, sec, sidx


def _make_router_sc(n_tokens):
    info = plsc.get_sparse_core_info()
    nc, ns = info.num_cores, info.num_subcores
    nw = nc * ns
    tpw = n_tokens // nw
    nblk = tpw // 16
    mesh = plsc.VectorSubcoreMesh(core_axis_name="c", subcore_axis_name="s")

    @functools.partial(
        pl.kernel,
        mesh=mesh,
        out_type=[
            jax.ShapeDtypeStruct((4, n_tokens), jnp.int32),
            jax.ShapeDtypeStruct((4, n_tokens), jnp.float32),
        ],
        scratch_types=[
            pltpu.VMEM((NE, tpw), jnp.float32),
            pltpu.VMEM((4, tpw), jnp.int32),
            pltpu.VMEM((4, tpw), jnp.float32),
        ],
        compiler_params=pltpu.CompilerParams(needs_layout_passes=False),
    )
    def router(scores_hbm, idx_hbm, probs_hbm, scores_v, idx_v, probs_v):
        wid = lax.axis_index("s") * nc + lax.axis_index("c")
        base = wid * tpw
        pltpu.sync_copy(scores_hbm.at[:, pl.ds(base, tpw)], scores_v)

        def step(t, carry):
            sl = pl.ds(t * 16, 16)
            cols = [scores_v[e, sl] for e in range(NE)]
            gs1, gi1, gs2, gi2 = _top2_of_8(cols[:G])
            es1, ei1, es2, ei2 = _top2_of_8(cols[G:])
            idxs = [gi1 * K_PER_G + ei1, gi1 * K_PER_G + ei2,
                    gi2 * K_PER_G + ei1, gi2 * K_PER_G + ei2]
            cs = [gs1 + es1, gs1 + es2, gs2 + es1, gs2 + es2]
            mx = jnp.maximum(jnp.maximum(cs[0], cs[1]),
                             jnp.maximum(cs[2], cs[3]))
            es = [jnp.exp(c - mx) for c in cs]
            tot = (es[0] + es[1]) + (es[2] + es[3])
            for j in range(4):
                idx_v[j, sl] = idxs[j]
                probs_v[j, sl] = es[j] / tot
            return carry

        lax.fori_loop(0, nblk, step, 0)
        pltpu.sync_copy(idx_v, idx_hbm.at[:, pl.ds(base, tpw)])
        pltpu.sync_copy(probs_v, probs_hbm.at[:, pl.ds(base, tpw)])

    return router


def _pack_body(i_ref, p_ref, oi_ref, op_ref):
    oi_ref[...] = i_ref[...].T
    op_ref[...] = p_ref[...].T


def _pack_tc(idx_t, probs_t):
    n = idx_t.shape[1]
    grid = (n // _TILE_P,)
    return pl.pallas_call(
        _pack_body,
        grid=grid,
        in_specs=[
            pl.BlockSpec((4, _TILE_P), lambda i: (0, i)),
            pl.BlockSpec((4, _TILE_P), lambda i: (0, i)),
        ],
        out_specs=[
            pl.BlockSpec((_TILE_P, 4), lambda i: (i, 0)),
            pl.BlockSpec((_TILE_P, 4), lambda i: (i, 0)),
        ],
        out_shape=[
            jax.ShapeDtypeStruct((n, 4), jnp.int32),
            jax.ShapeDtypeStruct((n, 4), jnp.float32),
        ],
    )(idx_t, probs_t)


def kernel(x, W1, W2):
    n_tokens = x.shape[0]
    scores_t = _scores_tc(x, W1, W2, 0, 1)
    router = _make_router_sc(n_tokens)
    idx_t, probs_t = router(scores_t)
    return (idx_t.T, probs_t.T)


# speedup vs baseline: 1.1632x; 1.0109x over previous
import functools

import jax
import jax.numpy as jnp
from jax import lax
from jax.experimental import pallas as pl
from jax.experimental.pallas import tpu as pltpu
from jax.experimental.pallas import tpu_sc as plsc

G = 8
K_PER_G = 8
G_ACTIVE = 2
K_PER_G_ACTIVE = 2
NE = G + K_PER_G

_TILE_M = 1024
_TILE_P = 2048

_DIMS = (((1,), (1,)), ((), ()))


def _mm_body(x1_ref, x2_ref, w1a_ref, w1b_ref, w2a_ref, w2b_ref, o_ref):
    sg = (lax.dot_general(w1a_ref[...], x1_ref[...], _DIMS,
                          preferred_element_type=jnp.float32)
          + lax.dot_general(w1b_ref[...], x2_ref[...], _DIMS,
                            preferred_element_type=jnp.float32))
    se = (lax.dot_general(w2a_ref[...], x1_ref[...], _DIMS,
                          preferred_element_type=jnp.float32)
          + lax.dot_general(w2b_ref[...], x2_ref[...], _DIMS,
                            preferred_element_type=jnp.float32))
    o_ref[:G, :] = sg
    o_ref[G:, :] = se


def _scores_tc(x, w1, w2, chunk, n_chunks):
    m, k = x.shape
    mc = m // n_chunks
    kh = k // 2
    grid = (mc // _TILE_M,)
    off = chunk * (mc // _TILE_M)
    return pl.pallas_call(
        _mm_body,
        grid=grid,
        in_specs=[
            pl.BlockSpec((_TILE_M, kh), lambda i: (i + off, 0)),
            pl.BlockSpec((_TILE_M, kh), lambda i: (i + off, 1)),
            pl.BlockSpec((G, kh), lambda i: (0, 0)),
            pl.BlockSpec((G, kh), lambda i: (0, 1)),
            pl.BlockSpec((K_PER_G, kh), lambda i: (0, 0)),
            pl.BlockSpec((K_PER_G, kh), lambda i: (0, 1)),
        ],
        out_specs=pl.BlockSpec((NE, _TILE_M), lambda i: (0, i)),
        out_shape=jax.ShapeDtypeStruct((NE, mc), jnp.float32),
    )(x, x, w1, w1, w2, w2)


def _top2_of_8(cols):
    best = cols[0]
    bidx = jnp.zeros((16,), jnp.int32)
    sec = jnp.full((16,), -jnp.inf, jnp.float32)
    sidx = jnp.zeros((16,), jnp.int32)
    for e in range(1, 8):
        ge = cols[e]
        ev = jnp.full((16,), e, jnp.int32)
        gt_b = ge > best
        gt_s = ge > sec
        sec = jnp.where(gt_b, best, jnp.where(gt_s, ge, sec))
        sidx = jnp.where(gt_b, bidx, jnp.where(gt_s, ev, sidx))
        best = jnp.where(gt_b, ge, best)
        bidx = jnp.where(gt_b, ev, bidx)
    return best, bidx, sec, sidx


def _make_router_sc(n_tokens):
    info = plsc.get_sparse_core_info()
    nc, ns = info.num_cores, info.num_subcores
    nw = nc * ns
    tpw = n_tokens // nw
    nblk = tpw // 16
    mesh = plsc.VectorSubcoreMesh(core_axis_name="c", subcore_axis_name="s")

    @functools.partial(
        pl.kernel,
        mesh=mesh,
        out_type=[
            jax.ShapeDtypeStruct((4, n_tokens), jnp.int32),
            jax.ShapeDtypeStruct((4, n_tokens), jnp.float32),
        ],
        scratch_types=[
            pltpu.VMEM((NE, tpw), jnp.float32),
            pltpu.VMEM((4, tpw), jnp.int32),
            pltpu.VMEM((4, tpw), jnp.float32),
        ],
        compiler_params=pltpu.CompilerParams(needs_layout_passes=False),
    )
    def router(scores_hbm, idx_hbm, probs_hbm, scores_v, idx_v, probs_v):
        wid = lax.axis_index("s") * nc + lax.axis_index("c")
        base = wid * tpw
        pltpu.sync_copy(scores_hbm.at[:, pl.ds(base, tpw)], scores_v)

        def step(t, carry):
            sl = pl.ds(t * 16, 16)
            cols = [scores_v[e, sl] for e in range(NE)]
            gs1, gi1, gs2, gi2 = _top2_of_8(cols[:G])
            es1, ei1, es2, ei2 = _top2_of_8(cols[G:])
            idxs = [gi1 * K_PER_G + ei1, gi1 * K_PER_G + ei2,
                    gi2 * K_PER_G + ei1, gi2 * K_PER_G + ei2]
            cs = [gs1 + es1, gs1 + es2, gs2 + es1, gs2 + es2]
            mx = jnp.maximum(jnp.maximum(cs[0], cs[1]),
                             jnp.maximum(cs[2], cs[3]))
            es = [jnp.exp(c - mx) for c in cs]
            tot = (es[0] + es[1]) + (es[2] + es[3])
            for j in range(4):
                idx_v[j, sl] = idxs[j]
                probs_v[j, sl] = es[j] / tot
            return carry

        lax.fori_loop(0, nblk, step, 0)
        pltpu.sync_copy(idx_v, idx_hbm.at[:, pl.ds(base, tpw)])
        pltpu.sync_copy(probs_v, probs_hbm.at[:, pl.ds(base, tpw)])

    return router


def _pack_body(i_ref, p_ref, oi_ref, op_ref):
    oi_ref[...] = i_ref[...].T
    op_ref[...] = p_ref[...].T


def _pack_tc(idx_t, probs_t):
    n = idx_t.shape[1]
    grid = (n // _TILE_P,)
    return pl.pallas_call(
        _pack_body,
        grid=grid,
        in_specs=[
            pl.BlockSpec((4, _TILE_P), lambda i: (0, i)),
            pl.BlockSpec((4, _TILE_P), lambda i: (0, i)),
        ],
        out_specs=[
            pl.BlockSpec((_TILE_P, 4), lambda i: (i, 0)),
            pl.BlockSpec((_TILE_P, 4), lambda i: (i, 0)),
        ],
        out_shape=[
            jax.ShapeDtypeStruct((n, 4), jnp.int32),
            jax.ShapeDtypeStruct((n, 4), jnp.float32),
        ],
    )(idx_t, probs_t)


def kernel(x, W1, W2):
    n_tokens = x.shape[0]
    scores_t = _scores_tc(x, W1, W2, 0, 1)
    router = _make_router_sc(n_tokens)
    idx_t, probs_t = router(scores_t)
    return (idx_t.T, probs_t.T)
